# baseline (speedup 1.0000x reference)
import jax
import jax.numpy as jnp
from jax.experimental import pallas as pl
from jax.experimental.pallas import tpu as pltpu


def _add_body(idx_ref, x_ref, w_ref, o_ref):
    o_ref[...] = (x_ref[...] + w_ref[...]) * 0.5


def kernel(x, blocks, W):
    B, NB, T, E = x.shape
    xf = x.reshape(B * NB, T, E)
    idx = blocks.reshape(-1).astype(jnp.int32)
    w3 = W.reshape(W.shape[0], 1, W.shape[1])

    out = pl.pallas_call(
        _add_body,
        grid_spec=pltpu.PrefetchScalarGridSpec(
            num_scalar_prefetch=1,
            grid=(B * NB,),
            in_specs=[
                pl.BlockSpec((1, T, E), lambda i, idx_ref: (i, 0, 0)),
                pl.BlockSpec((1, 1, E), lambda i, idx_ref: (idx_ref[i], 0, 0)),
            ],
            out_specs=pl.BlockSpec((1, T, E), lambda i, idx_ref: (i, 0, 0)),
        ),
        out_shape=jax.ShapeDtypeStruct((B * NB, T, E), x.dtype),
    )(idx, xf, w3)
    return out.reshape(B, NB * T, E)


# speedup vs baseline: 1.5255x; 1.5255x over previous
import jax
import jax.numpy as jnp
from jax.experimental import pallas as pl
from jax.experimental.pallas import tpu as pltpu


_ROWS_PER_STEP = 4


def _add_body(idx_ref, x_ref, w_ref, o_ref):
    i = pl.program_id(0)
    for j in range(_ROWS_PER_STEP):
        row = idx_ref[i * _ROWS_PER_STEP + j]
        o_ref[j] = (x_ref[j] + w_ref[pl.ds(row, 1), :]) * 0.5


def kernel(x, blocks, W):
    B, NB, T, E = x.shape
    R = _ROWS_PER_STEP
    xf = x.reshape(B * NB, T, E)
    idx = blocks.reshape(-1).astype(jnp.int32)

    out = pl.pallas_call(
        _add_body,
        grid_spec=pltpu.PrefetchScalarGridSpec(
            num_scalar_prefetch=1,
            grid=(B * NB // R,),
            in_specs=[
                pl.BlockSpec((R, T, E), lambda i, idx_ref: (i, 0, 0)),
                pl.BlockSpec((W.shape[0], E), lambda i, idx_ref: (0, 0)),
            ],
            out_specs=pl.BlockSpec((R, T, E), lambda i, idx_ref: (i, 0, 0)),
        ),
        out_shape=jax.ShapeDtypeStruct((B * NB, T, E), x.dtype),
    )(idx, xf, W)
    return out.reshape(B, NB * T, E)


# speedup vs baseline: 1.5544x; 1.0190x over previous
import jax
import jax.numpy as jnp
from jax.experimental import pallas as pl
from jax.experimental.pallas import tpu as pltpu


_ROWS_PER_STEP = 8


def _add_body(idx_ref, x_ref, w_ref, o_ref):
    i = pl.program_id(0)
    for j in range(_ROWS_PER_STEP):
        row = idx_ref[i * _ROWS_PER_STEP + j]
        o_ref[j] = (x_ref[j] + w_ref[pl.ds(row, 1), :]) * 0.5


def kernel(x, blocks, W):
    B, NB, T, E = x.shape
    R = _ROWS_PER_STEP
    xf = x.reshape(B * NB, T, E)
    idx = blocks.reshape(-1).astype(jnp.int32)

    out = pl.pallas_call(
        _add_body,
        grid_spec=pltpu.PrefetchScalarGridSpec(
            num_scalar_prefetch=1,
            grid=(B * NB // R,),
            in_specs=[
                pl.BlockSpec((R, T, E), lambda i, idx_ref: (i, 0, 0)),
                pl.BlockSpec((W.shape[0], E), lambda i, idx_ref: (0, 0)),
            ],
            out_specs=pl.BlockSpec((R, T, E), lambda i, idx_ref: (i, 0, 0)),
        ),
        out_shape=jax.ShapeDtypeStruct((B * NB, T, E), x.dtype),
    )(idx, xf, W)
    return out.reshape(B, NB * T, E)
